# Initial kernel scaffold; baseline (speedup 1.0000x reference)
import jax
import jax.numpy as jnp
from jax.experimental import pallas as pl


def kernel(x, edge_index, W1, b1, W2, b2, Wl, bl):
    raise NotImplementedError("write your pallas kernel here")


# speedup vs baseline: 8.5270x; 8.5270x over previous
import functools

import jax
import jax.numpy as jnp
from jax import lax
from jax.experimental import pallas as pl
from jax.experimental.pallas import tpu as pltpu
from jax.experimental.pallas import tpu_sc as plsc

N = 10000
D = 128
E = 320000

NC = 2
NS = 16
NW = NC * NS
CHUNK = 128
CHUNKS = 80
E_PAD = NW * CHUNKS * CHUNK
N_PAD = 10240
ROWS_PER_SUB = N_PAD // NS

@functools.cache
def _sc_kernels():
    mesh = plsc.VectorSubcoreMesh(core_axis_name="c", subcore_axis_name="s")

    @functools.partial(
        pl.kernel,
        out_type=jax.ShapeDtypeStruct((NC, N_PAD, D), jnp.float32),
        mesh=mesh,
        scratch_types=[
            pltpu.VMEM((CHUNKS, CHUNK), jnp.int32),
            pltpu.VMEM((CHUNK, D), jnp.float32),
            pltpu.VMEM_SHARED((N_PAD, D), jnp.float32),
        ],
    )
    def _deg_kernel(dst_hbm, ones_hbm, zeros_hbm, out_hbm, dstv, ones_v, accd):
        cid = lax.axis_index("c")
        sid = lax.axis_index("s")
        wid = sid * NC + cid
        row0 = sid * ROWS_PER_SUB
        pltpu.sync_copy(zeros_hbm.at[pl.ds(row0, ROWS_PER_SUB)],
                        accd.at[pl.ds(row0, ROWS_PER_SUB)])
        pltpu.sync_copy(ones_hbm, ones_v)
        pltpu.sync_copy(dst_hbm.at[wid], dstv)
        plsc.subcore_barrier()

        @pl.loop(0, CHUNKS)
        def _(c):
            pltpu.sync_copy(ones_v, accd.at[dstv.at[c]], add=True)

        plsc.subcore_barrier()
        pltpu.sync_copy(accd.at[pl.ds(row0, ROWS_PER_SUB)],
                        out_hbm.at[cid, pl.ds(row0, ROWS_PER_SUB)])

    @functools.partial(
        pl.kernel,
        out_type=jax.ShapeDtypeStruct((NC, N_PAD, D), jnp.float32),
        mesh=mesh,
        scratch_types=[
            pltpu.VMEM((CHUNKS, CHUNK), jnp.int32),
            pltpu.VMEM((CHUNKS, CHUNK), jnp.int32),
            pltpu.VMEM((CHUNK, D), jnp.float32),
            pltpu.VMEM_SHARED((N_PAD, D), jnp.float32),
            pltpu.SemaphoreType.DMA,
        ],
    )
    def _agg_kernel(y_hbm, src_hbm, dst_hbm, zeros_hbm, out_hbm,
                    srcv, dstv, rows, acc, sem):
        cid = lax.axis_index("c")
        sid = lax.axis_index("s")
        wid = sid * NC + cid
        row0 = sid * ROWS_PER_SUB
        pltpu.sync_copy(zeros_hbm.at[pl.ds(row0, ROWS_PER_SUB)],
                        acc.at[pl.ds(row0, ROWS_PER_SUB)])
        pltpu.sync_copy(src_hbm.at[wid], srcv)
        pltpu.sync_copy(dst_hbm.at[wid], dstv)
        plsc.subcore_barrier()

        @pl.loop(0, CHUNKS)
        def _(c):
            pltpu.async_copy(y_hbm.at[srcv.at[c]], rows, sem).wait()
            pltpu.sync_copy(rows, acc.at[dstv.at[c]], add=True)

        plsc.subcore_barrier()
        pltpu.sync_copy(acc.at[pl.ds(row0, ROWS_PER_SUB)],
                        out_hbm.at[cid, pl.ds(row0, ROWS_PER_SUB)])

    return _deg_kernel, _agg_kernel


_BM = 1024
_GRID = N_PAD // _BM


def _mm_body(x_ref, w_ref, o_ref):
    o_ref[...] = jnp.dot(x_ref[...], w_ref[...],
                         preferred_element_type=jnp.float32)


def _tc_matmul(x, w):
    return pl.pallas_call(
        _mm_body,
        grid=(_GRID,),
        in_specs=[
            pl.BlockSpec((_BM, D), lambda i: (i, 0)),
            pl.BlockSpec((D, D), lambda i: (0, 0)),
        ],
        out_specs=pl.BlockSpec((_BM, D), lambda i: (i, 0)),
        out_shape=jax.ShapeDtypeStruct((N_PAD, D), jnp.float32),
    )(x, w)


def _scale_body(degp_ref, xw_ref, dinv_ref, y_ref):
    deg = degp_ref[0, :, 0:1] + degp_ref[1, :, 0:1] + 1.0
    dinv = lax.rsqrt(deg)
    dinv_ref[...] = jnp.broadcast_to(dinv, dinv_ref.shape)
    y_ref[...] = dinv * xw_ref[...]


def _tc_scale(deg_partials, xw):
    return pl.pallas_call(
        _scale_body,
        grid=(_GRID,),
        in_specs=[
            pl.BlockSpec((NC, _BM, D), lambda i: (0, i, 0)),
            pl.BlockSpec((_BM, D), lambda i: (i, 0)),
        ],
        out_specs=[
            pl.BlockSpec((_BM, D), lambda i: (i, 0)),
            pl.BlockSpec((_BM, D), lambda i: (i, 0)),
        ],
        out_shape=[
            jax.ShapeDtypeStruct((N_PAD, D), jnp.float32),
            jax.ShapeDtypeStruct((N_PAD, D), jnp.float32),
        ],
    )(deg_partials, xw)


def _mid_body(p_ref, y_ref, dinv_ref, b_ref, w_ref, y2_ref):
    agg = p_ref[0] + p_ref[1] + y_ref[...]
    h = jnp.maximum(dinv_ref[...] * agg + b_ref[...], 0.0)
    y2_ref[...] = dinv_ref[...] * jnp.dot(h, w_ref[...],
                                          preferred_element_type=jnp.float32)


def _tc_mid(partials, y, dinv, b, w):
    return pl.pallas_call(
        _mid_body,
        grid=(_GRID,),
        in_specs=[
            pl.BlockSpec((NC, _BM, D), lambda i: (0, i, 0)),
            pl.BlockSpec((_BM, D), lambda i: (i, 0)),
            pl.BlockSpec((_BM, D), lambda i: (i, 0)),
            pl.BlockSpec((1, D), lambda i: (0, 0)),
            pl.BlockSpec((D, D), lambda i: (0, 0)),
        ],
        out_specs=pl.BlockSpec((_BM, D), lambda i: (i, 0)),
        out_shape=jax.ShapeDtypeStruct((N_PAD, D), jnp.float32),
    )(partials, y, dinv, b, w)


def _final_body(p_ref, y_ref, dinv_ref, b_ref, w_ref, bl_ref, o_ref):
    agg = p_ref[0] + p_ref[1] + y_ref[...]
    h = jnp.maximum(dinv_ref[...] * agg + b_ref[...], 0.0)
    o_ref[...] = jnp.dot(h, w_ref[...],
                         preferred_element_type=jnp.float32) + bl_ref[...]


def _tc_final(partials, y, dinv, b, w, bl):
    return pl.pallas_call(
        _final_body,
        grid=(_GRID,),
        in_specs=[
            pl.BlockSpec((NC, _BM, D), lambda i: (0, i, 0)),
            pl.BlockSpec((_BM, D), lambda i: (i, 0)),
            pl.BlockSpec((_BM, D), lambda i: (i, 0)),
            pl.BlockSpec((1, D), lambda i: (0, 0)),
            pl.BlockSpec((D, D), lambda i: (0, 0)),
            pl.BlockSpec((1, D), lambda i: (0, 0)),
        ],
        out_specs=pl.BlockSpec((_BM, D), lambda i: (i, 0)),
        out_shape=jax.ShapeDtypeStruct((N_PAD, D), jnp.float32),
    )(partials, y, dinv, b, w, bl)


def kernel(x, edge_index, W1, b1, W2, b2, Wl, bl):
    src = edge_index[0].astype(jnp.int32)
    dst = edge_index[1].astype(jnp.int32)
    pad = E_PAD - E
    src_p = jnp.concatenate([src, jnp.zeros((pad,), jnp.int32)])
    dst_p = jnp.concatenate([dst, jnp.full((pad,), N_PAD - 1, jnp.int32)])
    src3 = src_p.reshape(NW, CHUNKS, CHUNK)
    dst3 = dst_p.reshape(NW, CHUNKS, CHUNK)

    x_p = jnp.pad(x, ((0, N_PAD - N), (0, 0)))
    onesD = jnp.ones((CHUNK, D), jnp.float32)
    zerosD = jnp.zeros((N_PAD, D), jnp.float32)
    b1r = b1.reshape(1, D)
    b2r = b2.reshape(1, D)
    blr = bl.reshape(1, D)

    _deg_kernel, _agg_kernel = _sc_kernels()

    deg_partials = _deg_kernel(dst3, onesD, zerosD)
    xw1 = _tc_matmul(x_p, W1)
    dinv, y1 = _tc_scale(deg_partials, xw1)
    p1 = _agg_kernel(y1, src3, dst3, zerosD)
    y2 = _tc_mid(p1, y1, dinv, b1r, W2)
    p2 = _agg_kernel(y2, src3, dst3, zerosD)
    out = _tc_final(p2, y2, dinv, b2r, Wl, blr)
    return out[:N]


# speedup vs baseline: 9.0937x; 1.0665x over previous
import functools

import jax
import jax.numpy as jnp
from jax import lax
from jax.experimental import pallas as pl
from jax.experimental.pallas import tpu as pltpu
from jax.experimental.pallas import tpu_sc as plsc

N = 10000
D = 128
E = 320000

NC = 2
NS = 16
NW = NC * NS
CHUNK = 128
CHUNKS = 80
GROUPS = 5
GCHUNK = CHUNKS // GROUPS
E_PAD = NW * CHUNKS * CHUNK
N_PAD = 10240
ROWS_PER_SUB = N_PAD // NS

@functools.cache
def _sc_kernels():
    mesh = plsc.VectorSubcoreMesh(core_axis_name="c", subcore_axis_name="s")

    @functools.partial(
        pl.kernel,
        out_type=jax.ShapeDtypeStruct((NC, N_PAD, D), jnp.float32),
        mesh=mesh,
        scratch_types=[
            pltpu.VMEM((CHUNKS, CHUNK), jnp.int32),
            pltpu.VMEM((CHUNK, D), jnp.float32),
            pltpu.VMEM_SHARED((N_PAD, D), jnp.float32),
        ],
    )
    def _deg_kernel(dst_hbm, ones_hbm, zeros_hbm, out_hbm, dstv, ones_v, accd):
        cid = lax.axis_index("c")
        sid = lax.axis_index("s")
        wid = sid * NC + cid
        row0 = sid * ROWS_PER_SUB
        pltpu.sync_copy(zeros_hbm.at[pl.ds(row0, ROWS_PER_SUB)],
                        accd.at[pl.ds(row0, ROWS_PER_SUB)])
        pltpu.sync_copy(ones_hbm, ones_v)
        pltpu.sync_copy(dst_hbm.at[wid], dstv)
        plsc.subcore_barrier()

        @pl.loop(0, CHUNKS)
        def _(c):
            pltpu.sync_copy(ones_v, accd.at[dstv.at[c]], add=True)

        plsc.subcore_barrier()
        pltpu.sync_copy(accd.at[pl.ds(row0, ROWS_PER_SUB)],
                        out_hbm.at[cid, pl.ds(row0, ROWS_PER_SUB)])

    @functools.partial(
        pl.kernel,
        out_type=jax.ShapeDtypeStruct((NC, N_PAD, D), jnp.float32),
        mesh=mesh,
        scratch_types=[
            pltpu.VMEM((GCHUNK, CHUNK), jnp.int32),
            pltpu.VMEM((GCHUNK, CHUNK), jnp.int32),
            pltpu.VMEM((GCHUNK, CHUNK), jnp.int32),
            pltpu.VMEM((GCHUNK, CHUNK), jnp.int32),
            pltpu.VMEM((CHUNK, D), jnp.float32),
            pltpu.VMEM((CHUNK, D), jnp.float32),
            pltpu.VMEM_SHARED((N_PAD, D), jnp.float32),
            pltpu.SemaphoreType.DMA,
            pltpu.SemaphoreType.DMA,
            pltpu.SemaphoreType.DMA,
        ],
    )
    def _agg_kernel(y_hbm, src_hbm, dst_hbm, zeros_hbm, out_hbm,
                    srcA, srcB, dstA, dstB, rows0, rows1, acc,
                    sem0, sem1, semi):
        cid = lax.axis_index("c")
        sid = lax.axis_index("s")
        wid = sid * NC + cid
        row0 = sid * ROWS_PER_SUB
        pltpu.sync_copy(zeros_hbm.at[pl.ds(row0, ROWS_PER_SUB)],
                        acc.at[pl.ds(row0, ROWS_PER_SUB)])
        pltpu.sync_copy(src_hbm.at[wid, pl.ds(0, GCHUNK)], srcA)
        pltpu.sync_copy(dst_hbm.at[wid, pl.ds(0, GCHUNK)], dstA)
        plsc.subcore_barrier()

        pltpu.make_async_copy(y_hbm.at[srcA.at[0]], rows0, sem0).start()
        for g in range(GROUPS):
            s_cur, d_cur = (srcA, dstA) if g % 2 == 0 else (srcB, dstB)
            s_nxt, d_nxt = (srcB, dstB) if g % 2 == 0 else (srcA, dstA)
            if g + 1 < GROUPS:
                nxt = pl.ds((g + 1) * GCHUNK, GCHUNK)
                pltpu.make_async_copy(src_hbm.at[wid, nxt], s_nxt,
                                      semi).start()
                pltpu.make_async_copy(dst_hbm.at[wid, nxt], d_nxt,
                                      semi).start()

            @pl.loop(0, GCHUNK // 2)
            def _(h, s_cur=s_cur, d_cur=d_cur):
                c = h * 2
                pltpu.make_async_copy(y_hbm.at[s_cur.at[c]], rows0,
                                      sem0).wait()
                pltpu.make_async_copy(y_hbm.at[s_cur.at[c + 1]], rows1,
                                      sem1).start()
                pltpu.sync_copy(rows0, acc.at[d_cur.at[c]], add=True)
                pltpu.make_async_copy(y_hbm.at[s_cur.at[c + 1]], rows1,
                                      sem1).wait()

                @pl.when(h < GCHUNK // 2 - 1)
                def _():
                    pltpu.make_async_copy(y_hbm.at[s_cur.at[c + 2]], rows0,
                                          sem0).start()

                pltpu.sync_copy(rows1, acc.at[d_cur.at[c + 1]], add=True)

            if g + 1 < GROUPS:
                nxt = pl.ds((g + 1) * GCHUNK, GCHUNK)
                pltpu.make_async_copy(src_hbm.at[wid, nxt], s_nxt,
                                      semi).wait()
                pltpu.make_async_copy(dst_hbm.at[wid, nxt], d_nxt,
                                      semi).wait()
                pltpu.make_async_copy(y_hbm.at[s_nxt.at[0]], rows0,
                                      sem0).start()

        plsc.subcore_barrier()
        pltpu.sync_copy(acc.at[pl.ds(row0, ROWS_PER_SUB)],
                        out_hbm.at[cid, pl.ds(row0, ROWS_PER_SUB)])

    return _deg_kernel, _agg_kernel


_BM = 1024
_GRID = N_PAD // _BM


def _mm_body(x_ref, w_ref, o_ref):
    o_ref[...] = jnp.dot(x_ref[...], w_ref[...],
                         preferred_element_type=jnp.float32)


def _tc_matmul(x, w):
    return pl.pallas_call(
        _mm_body,
        grid=(_GRID,),
        in_specs=[
            pl.BlockSpec((_BM, D), lambda i: (i, 0)),
            pl.BlockSpec((D, D), lambda i: (0, 0)),
        ],
        out_specs=pl.BlockSpec((_BM, D), lambda i: (i, 0)),
        out_shape=jax.ShapeDtypeStruct((N_PAD, D), jnp.float32),
    )(x, w)


def _scale_body(degp_ref, xw_ref, dinv_ref, y_ref):
    deg = degp_ref[0, :, 0:1] + degp_ref[1, :, 0:1] + 1.0
    dinv = lax.rsqrt(deg)
    dinv_ref[...] = jnp.broadcast_to(dinv, dinv_ref.shape)
    y_ref[...] = dinv * xw_ref[...]


def _tc_scale(deg_partials, xw):
    return pl.pallas_call(
        _scale_body,
        grid=(_GRID,),
        in_specs=[
            pl.BlockSpec((NC, _BM, D), lambda i: (0, i, 0)),
            pl.BlockSpec((_BM, D), lambda i: (i, 0)),
        ],
        out_specs=[
            pl.BlockSpec((_BM, D), lambda i: (i, 0)),
            pl.BlockSpec((_BM, D), lambda i: (i, 0)),
        ],
        out_shape=[
            jax.ShapeDtypeStruct((N_PAD, D), jnp.float32),
            jax.ShapeDtypeStruct((N_PAD, D), jnp.float32),
        ],
    )(deg_partials, xw)


def _mid_body(p_ref, y_ref, dinv_ref, b_ref, w_ref, y2_ref):
    agg = p_ref[0] + p_ref[1] + y_ref[...]
    h = jnp.maximum(dinv_ref[...] * agg + b_ref[...], 0.0)
    y2_ref[...] = dinv_ref[...] * jnp.dot(h, w_ref[...],
                                          preferred_element_type=jnp.float32)


def _tc_mid(partials, y, dinv, b, w):
    return pl.pallas_call(
        _mid_body,
        grid=(_GRID,),
        in_specs=[
            pl.BlockSpec((NC, _BM, D), lambda i: (0, i, 0)),
            pl.BlockSpec((_BM, D), lambda i: (i, 0)),
            pl.BlockSpec((_BM, D), lambda i: (i, 0)),
            pl.BlockSpec((1, D), lambda i: (0, 0)),
            pl.BlockSpec((D, D), lambda i: (0, 0)),
        ],
        out_specs=pl.BlockSpec((_BM, D), lambda i: (i, 0)),
        out_shape=jax.ShapeDtypeStruct((N_PAD, D), jnp.float32),
    )(partials, y, dinv, b, w)


def _final_body(p_ref, y_ref, dinv_ref, b_ref, w_ref, bl_ref, o_ref):
    agg = p_ref[0] + p_ref[1] + y_ref[...]
    h = jnp.maximum(dinv_ref[...] * agg + b_ref[...], 0.0)
    o_ref[...] = jnp.dot(h, w_ref[...],
                         preferred_element_type=jnp.float32) + bl_ref[...]


def _tc_final(partials, y, dinv, b, w, bl):
    return pl.pallas_call(
        _final_body,
        grid=(_GRID,),
        in_specs=[
            pl.BlockSpec((NC, _BM, D), lambda i: (0, i, 0)),
            pl.BlockSpec((_BM, D), lambda i: (i, 0)),
            pl.BlockSpec((_BM, D), lambda i: (i, 0)),
            pl.BlockSpec((1, D), lambda i: (0, 0)),
            pl.BlockSpec((D, D), lambda i: (0, 0)),
            pl.BlockSpec((1, D), lambda i: (0, 0)),
        ],
        out_specs=pl.BlockSpec((_BM, D), lambda i: (i, 0)),
        out_shape=jax.ShapeDtypeStruct((N_PAD, D), jnp.float32),
    )(partials, y, dinv, b, w, bl)


def kernel(x, edge_index, W1, b1, W2, b2, Wl, bl):
    src = edge_index[0].astype(jnp.int32)
    dst = edge_index[1].astype(jnp.int32)
    pad = E_PAD - E
    src_p = jnp.concatenate([src, jnp.zeros((pad,), jnp.int32)])
    dst_p = jnp.concatenate([dst, jnp.full((pad,), N_PAD - 1, jnp.int32)])
    src3 = src_p.reshape(NW, CHUNKS, CHUNK)
    dst3 = dst_p.reshape(NW, CHUNKS, CHUNK)

    x_p = jnp.pad(x, ((0, N_PAD - N), (0, 0)))
    onesD = jnp.ones((CHUNK, D), jnp.float32)
    zerosD = jnp.zeros((N_PAD, D), jnp.float32)
    b1r = b1.reshape(1, D)
    b2r = b2.reshape(1, D)
    blr = bl.reshape(1, D)

    _deg_kernel, _agg_kernel = _sc_kernels()

    deg_partials = _deg_kernel(dst3, onesD, zerosD)
    xw1 = _tc_matmul(x_p, W1)
    dinv, y1 = _tc_scale(deg_partials, xw1)
    p1 = _agg_kernel(y1, src3, dst3, zerosD)
    y2 = _tc_mid(p1, y1, dinv, b1r, W2)
    p2 = _agg_kernel(y2, src3, dst3, zerosD)
    out = _tc_final(p2, y2, dinv, b2r, Wl, blr)
    return out[:N]
